# baseline (speedup 1.0000x reference)
import functools

import jax
import jax.numpy as jnp
from jax import lax
from jax.experimental import pallas as pl
from jax.experimental.pallas import tpu as pltpu
from jax.experimental.pallas import tpu_sc as plsc

N_NODES = 50000
N_EDGES = 1600000
CUTOFF = 10.0
NBASIS = 16

NW = 32
C = 128
EPW = 50176
NCH = EPW // C
NE_PAD = NW * EPW
N_PAD = 50176
DUMMY = N_NODES
NER = NE_PAD // 128
SQRT3 = 1.7320508075688772


_SC_PARAMS = pltpu.CompilerParams(use_tc_tiling_on_sc=False)


def _sc_gather(T, src_r, dst_r):
    mesh = plsc.VectorSubcoreMesh(core_axis_name="c", subcore_axis_name="s")

    @functools.partial(
        pl.kernel,
        out_type=(jax.ShapeDtypeStruct((NE_PAD, 8), jnp.float32),
                  jax.ShapeDtypeStruct((NE_PAD, 8), jnp.float32)),
        mesh=mesh,
        compiler_params=_SC_PARAMS,
        scratch_types=[pltpu.VMEM((NCH, C), jnp.int32),
                       pltpu.VMEM((NCH, C), jnp.int32),
                       pltpu.VMEM((C, 8), jnp.float32),
                       pltpu.VMEM((C, 8), jnp.float32),
                       pltpu.SemaphoreType.DMA,
                       pltpu.SemaphoreType.DMA],
    )
    def k(T_hbm, si_hbm, di_hbm, gs_hbm, gd_hbm,
          si_v, di_v, rs_v, rd_v, sem1, sem2):
        cid = lax.axis_index("c")
        sid = lax.axis_index("s")
        wid = cid * 16 + sid
        pltpu.sync_copy(si_hbm.at[wid], si_v)
        pltpu.sync_copy(di_hbm.at[wid], di_v)
        base = wid * EPW

        @pl.loop(0, NCH)
        def _(j):
            a = pltpu.async_copy(T_hbm.at[si_v.at[j]], rs_v, sem1)
            b = pltpu.async_copy(T_hbm.at[di_v.at[j]], rd_v, sem2)
            a.wait()
            b.wait()
            off = base + j * C
            pltpu.sync_copy(rs_v, gs_hbm.at[pl.ds(off, C)])
            pltpu.sync_copy(rd_v, gd_hbm.at[pl.ds(off, C)])

    return k(T, src_r, dst_r)


def _sc_scatter(msgT, dst_r, zblk):
    mesh = plsc.VectorSubcoreMesh(core_axis_name="c", subcore_axis_name="s")
    rows_per_sub = N_PAD // 16

    @functools.partial(
        pl.kernel,
        out_type=jax.ShapeDtypeStruct((2, N_PAD, 8), jnp.float32),
        mesh=mesh,
        compiler_params=_SC_PARAMS,
        scratch_types=[pltpu.VMEM((NCH, C), jnp.int32),
                       pltpu.VMEM((C, 8), jnp.float32),
                       pltpu.VMEM_SHARED((N_PAD, 8), jnp.float32)],
    )
    def k(msg_hbm, di_hbm, z_hbm, out_hbm, di_v, rows_v, acc):
        cid = lax.axis_index("c")
        sid = lax.axis_index("s")
        wid = cid * 16 + sid
        pltpu.sync_copy(di_hbm.at[wid], di_v)
        pltpu.sync_copy(z_hbm, acc.at[pl.ds(sid * rows_per_sub, rows_per_sub)])
        plsc.subcore_barrier()
        base = wid * EPW

        @pl.loop(0, NCH)
        def _(j):
            pltpu.sync_copy(msg_hbm.at[pl.ds(base + j * C, C)], rows_v)
            pltpu.sync_copy(rows_v, acc.at[di_v.at[j]], add=True)

        plsc.subcore_barrier()

        @pl.when(sid == 0)
        def _():
            pltpu.sync_copy(acc, out_hbm.at[cid])

    return k(msgT, dst_r, zblk)


def _edge_body(gs_ref, gd_ref, alpha_ref, msg_ref, erbf_ref, ersh_ref):
    gs = jnp.transpose(gs_ref[...], (0, 2, 1))
    gd = jnp.transpose(gd_ref[...], (0, 2, 1))
    gs = [gs[:, c, :] for c in range(8)]
    gd = [gd[:, c, :] for c in range(3)]
    vx = gd[0] - gs[0]
    vy = gd[1] - gs[1]
    vz = gd[2] - gs[2]
    d2 = vx * vx + vy * vy + vz * vz
    dist = jnp.sqrt(d2 + 1e-12)
    invd = 1.0 / dist
    ux = vx * invd
    uy = vy * invd
    uz = vz * invd
    t = dist * (1.0 / CUTOFF)
    t2 = t * t
    t3 = t2 * t
    t6 = t3 * t3
    t7 = t6 * t
    t8 = t7 * t
    fc = 1.0 - 28.0 * t6 + 48.0 * t7 - 21.0 * t8
    fc = jnp.where(t < 1.0, fc, 0.0)
    a0 = alpha_ref[0]
    a1 = alpha_ref[1]
    rad0 = jnp.exp(-a0 * d2) * fc
    rad1 = jnp.exp(-a1 * d2) * fc
    g1 = SQRT3 * rad1
    z = jnp.zeros_like(ux)
    msg = jnp.stack([gs[3] * rad0, gs[4] * (g1 * ux), gs[5] * (g1 * uy),
                     gs[6] * (g1 * uz), z, z, z, z], axis=1)
    msg_ref[...] = jnp.transpose(msg, (0, 2, 1))
    theta = dist * (jnp.pi / CUTOFF)
    s1 = jnp.sin(theta)
    c2 = 2.0 * jnp.cos(theta)
    pf = jnp.sqrt(2.0 / CUTOFF) * fc * invd
    sm2 = jnp.zeros_like(s1)
    sm1 = s1
    eb = [sm1 * pf]
    for n in range(1, NBASIS):
        sn = c2 * sm1 - sm2
        sm2 = sm1
        sm1 = sn
        eb.append(sn * pf)
    erbf_ref[...] = jnp.transpose(jnp.stack(eb, axis=1), (0, 2, 1))
    ersh = jnp.stack([jnp.ones_like(ux), -SQRT3 * ux, -SQRT3 * uy,
                      -SQRT3 * uz], axis=1)
    ersh_ref[...] = jnp.transpose(ersh, (0, 2, 1))


NERR = N_EDGES // 128


def _tc_edge(gs_r, gd_r, gto_alpha):
    BR = 25
    grid = (NERR // BR,)
    return pl.pallas_call(
        _edge_body,
        grid=grid,
        in_specs=[
            pl.BlockSpec((BR, 128, 8), lambda i: (i, 0, 0)),
            pl.BlockSpec((BR, 128, 8), lambda i: (i, 0, 0)),
            pl.BlockSpec(memory_space=pltpu.SMEM),
        ],
        out_specs=[
            pl.BlockSpec((BR, 128, 8), lambda i: (i, 0, 0)),
            pl.BlockSpec((BR, 128, NBASIS), lambda i: (i, 0, 0)),
            pl.BlockSpec((BR, 128, 4), lambda i: (i, 0, 0)),
        ],
        out_shape=[
            jax.ShapeDtypeStruct((NER, 128, 8), jnp.float32),
            jax.ShapeDtypeStruct((NERR, 128, NBASIS), jnp.float32),
            jax.ShapeDtypeStruct((NERR, 128, 4), jnp.float32),
        ],
    )(gs_r, gd_r, gto_alpha)


def _br(a):
    return a.astype(jnp.bfloat16).astype(jnp.float32)


def _node_math(sph, w_ref, W01_ref, W11_ref, W02_ref, W12_ref):
    s = sph[:, 0:1]
    vx = sph[:, 1:2]
    vy = sph[:, 2:3]
    vz = sph[:, 3:4]
    w0 = w_ref[0]
    w1 = w_ref[1]
    w2 = w_ref[2]
    w3 = w_ref[3]
    o0a = w0 * s * s
    o0b = (w1 / SQRT3) * (vx * vx + vy * vy + vz * vz)
    W01 = _br(W01_ref[...] * (1.0 / jnp.sqrt(2.0)))
    ns = _br(o0a) * W01[0:1, :] + _br(o0b) * W01[1:2, :]
    W11 = _br(W11_ref[...] * (1.0 / jnp.sqrt(2.0)))
    sv = s
    nvx = _br(w2 * sv * vx) * W11[0:1, :] + _br(w3 * sv * vx) * W11[1:2, :]
    nvy = _br(w2 * sv * vy) * W11[0:1, :] + _br(w3 * sv * vy) * W11[1:2, :]
    nvz = _br(w2 * sv * vz) * W11[0:1, :] + _br(w3 * sv * vz) * W11[1:2, :]
    ns = jax.nn.sigmoid(ns)
    vnorm = jnp.sqrt(nvx * nvx + nvy * nvy + nvz * nvz + 1e-12)
    gate = jax.nn.sigmoid(vnorm)
    nvx = nvx * gate
    nvy = nvy * gate
    nvz = nvz * gate
    bf = jnp.bfloat16
    f32 = jnp.float32
    W02 = (W02_ref[...] * (1.0 / jnp.sqrt(128.0))).astype(bf)
    ns2 = jnp.dot(ns.astype(bf), W02, preferred_element_type=f32)
    W12 = (W12_ref[...] * (1.0 / 8.0)).astype(bf)
    nvx2 = jnp.dot(nvx.astype(bf), W12, preferred_element_type=f32)
    nvy2 = jnp.dot(nvy.astype(bf), W12, preferred_element_type=f32)
    nvz2 = jnp.dot(nvz.astype(bf), W12, preferred_element_type=f32)
    return ns2, nvx2, nvy2, nvz2


BN = 2000


def _stats_body(sph_ref, w_ref, W01_ref, W11_ref, W02_ref, W12_ref, st_ref):
    sph = sph_ref[0] + sph_ref[1]
    ns2, nvx2, nvy2, nvz2 = _node_math(sph, w_ref, W01_ref, W11_ref,
                                       W02_ref, W12_ref)
    ssum = jnp.sum(ns2, axis=0).reshape(1, 128)
    ssq = jnp.sum(ns2 * ns2, axis=0).reshape(1, 128)
    vn2 = jnp.sum(nvx2 * nvx2 + nvy2 * nvy2 + nvz2 * nvz2, axis=0)
    vn2 = jnp.concatenate([vn2, jnp.zeros((64,), jnp.float32)]).reshape(1, 128)
    contrib = jnp.concatenate(
        [ssum, ssq, vn2, jnp.zeros((5, 128), jnp.float32)], axis=0)

    @pl.when(pl.program_id(0) == 0)
    def _():
        st_ref[...] = jnp.zeros_like(st_ref)

    st_ref[...] += contrib


def _norm_body(sph_ref, st_ref, w_ref, W01_ref, W11_ref, W02_ref, W12_ref,
               gs_ref, bs_ref, gv_ref, ns_ref, nvx_ref, nvy_ref, nvz_ref):
    sph = sph_ref[0] + sph_ref[1]
    ns2, nvx2, nvy2, nvz2 = _node_math(sph, w_ref, W01_ref, W11_ref,
                                       W02_ref, W12_ref)
    st = st_ref[...]
    inv_n = 1.0 / N_NODES
    mean = st[0:1, :] * inv_n
    var = st[1:2, :] * inv_n - mean * mean
    scale = gs_ref[...] / jnp.sqrt(var + 1e-5)
    ns_ref[...] = (ns2 - mean) * scale + bs_ref[...]
    vn2m = st[2:3, 0:64] * inv_n
    vfac = gv_ref[...] / jnp.sqrt(vn2m + 1e-5)
    nvx_ref[...] = nvx2 * vfac
    nvy_ref[...] = nvy2 * vfac
    nvz_ref[...] = nvz2 * vfac


def _tc_node(parts, w_self, W0_1, W1_1, W0_2, W1_2, gamma_s, beta_s, gamma_v):
    nb = N_NODES // BN
    wspec = [
        pl.BlockSpec(memory_space=pltpu.SMEM),
        pl.BlockSpec((2, 128), lambda i: (0, 0)),
        pl.BlockSpec((2, 64), lambda i: (0, 0)),
        pl.BlockSpec((128, 128), lambda i: (0, 0)),
        pl.BlockSpec((64, 64), lambda i: (0, 0)),
    ]
    sph_spec = pl.BlockSpec((2, BN, 8), lambda i: (0, i, 0))
    stats = pl.pallas_call(
        _stats_body,
        grid=(nb,),
        in_specs=[sph_spec] + wspec,
        out_specs=pl.BlockSpec((8, 128), lambda i: (0, 0)),
        out_shape=jax.ShapeDtypeStruct((8, 128), jnp.float32),
    )(parts, w_self, W0_1, W1_1, W0_2, W1_2)
    ns, nvx, nvy, nvz = pl.pallas_call(
        _norm_body,
        grid=(nb,),
        in_specs=[sph_spec, pl.BlockSpec((8, 128), lambda i: (0, 0))] + wspec
        + [pl.BlockSpec((1, 128), lambda i: (0, 0)),
           pl.BlockSpec((1, 128), lambda i: (0, 0)),
           pl.BlockSpec((1, 64), lambda i: (0, 0))],
        out_specs=[
            pl.BlockSpec((BN, 128), lambda i: (i, 0)),
            pl.BlockSpec((BN, 64), lambda i: (i, 0)),
            pl.BlockSpec((BN, 64), lambda i: (i, 0)),
            pl.BlockSpec((BN, 64), lambda i: (i, 0)),
        ],
        out_shape=[
            jax.ShapeDtypeStruct((N_NODES, 128), jnp.float32),
            jax.ShapeDtypeStruct((N_NODES, 64), jnp.float32),
            jax.ShapeDtypeStruct((N_NODES, 64), jnp.float32),
            jax.ShapeDtypeStruct((N_NODES, 64), jnp.float32),
        ],
    )(parts, stats, w_self, W0_1, W1_1, W0_2, W1_2,
      gamma_s.reshape(1, 128), beta_s.reshape(1, 128), gamma_v.reshape(1, 64))
    return ns, nvx, nvy, nvz


def kernel(x, pos, edge_index, w_self, W0_1, W1_1, W0_2, W1_2,
           gamma_s, beta_s, gamma_v, gto_alpha):
    f32 = jnp.float32
    pos_p = pos[:, jnp.array([1, 2, 0])]
    T = jnp.zeros((N_PAD, 8), f32)
    T = T.at[:N_NODES, 0:3].set(pos_p).at[:N_NODES, 3:7].set(x)
    npad = NE_PAD - N_EDGES
    src = jnp.concatenate([edge_index[0], jnp.zeros((npad,), jnp.int32)])
    dst = jnp.concatenate([edge_index[1],
                           jnp.full((npad,), DUMMY, jnp.int32)])
    src_r = src.reshape(NW, NCH, C)
    dst_r = dst.reshape(NW, NCH, C)

    gs, gd = _sc_gather(T, src_r, dst_r)

    gs_r = gs.reshape(NER, 128, 8)
    gd_r = gd.reshape(NER, 128, 8)
    msg_r, erbf_r, ersh_r = _tc_edge(gs_r, gd_r, gto_alpha)

    erbf = erbf_r.reshape(N_EDGES, NBASIS)
    ersh = ersh_r.reshape(N_EDGES, 4)
    msgT = msg_r.reshape(NE_PAD, 8)

    zblk = jnp.zeros((N_PAD // 16, 8), f32)
    parts = _sc_scatter(msgT, dst_r, zblk)

    ns, nvx, nvy, nvz = _tc_node(parts, w_self, W0_1, W1_1, W0_2, W1_2,
                                 gamma_s, beta_s, gamma_v)
    nv = jnp.stack([nvx, nvy, nvz], axis=-1).reshape(N_NODES, 192)
    node = jnp.concatenate([ns, nv], axis=1)
    return node, erbf, ersh


# speedup vs baseline: 2.1865x; 2.1865x over previous
import functools

import jax
import jax.numpy as jnp
from jax import lax
from jax.experimental import pallas as pl
from jax.experimental.pallas import tpu as pltpu
from jax.experimental.pallas import tpu_sc as plsc

N_NODES = 50000
N_EDGES = 1600000
CUTOFF = 10.0
NBASIS = 16

NW = 32
C = 128
EPW = 50176
NCH = EPW // C
NSUP = NCH // 8
NE_PAD = NW * EPW
N_PAD = 50176
DUMMY = N_NODES
NERR = N_EDGES // 128
SQRT3 = 1.7320508075688772

_SC_PARAMS = pltpu.CompilerParams(use_tc_tiling_on_sc=False,
                                  needs_layout_passes=False)


def _sc_edge(T, src_r, dst_r, gto_alpha, zblk):
    mesh = plsc.VectorSubcoreMesh(core_axis_name="c", subcore_axis_name="s")
    rows_per_sub = N_PAD // 16
    i32 = jnp.int32
    f32 = jnp.float32

    @functools.partial(
        pl.kernel,
        out_type=(jax.ShapeDtypeStruct((4, NE_PAD), f32),
                  jax.ShapeDtypeStruct((2, N_PAD, 8), f32)),
        mesh=mesh,
        compiler_params=_SC_PARAMS,
        scratch_types=[pltpu.VMEM((8, C), i32),
                       pltpu.VMEM((8, C), i32),
                       pltpu.VMEM((C, 8), f32),
                       pltpu.VMEM((C, 8), f32),
                       pltpu.VMEM((C, 8), f32),
                       pltpu.VMEM((1024,), f32),
                       pltpu.VMEM((1024,), f32),
                       pltpu.VMEM((1024,), f32),
                       pltpu.VMEM((1024,), f32),
                       pltpu.VMEM((32,), f32),
                       pltpu.VMEM_SHARED((N_PAD, 8), f32),
                       pltpu.SemaphoreType.DMA,
                       pltpu.SemaphoreType.DMA],
    )
    def k(T_hbm, si_hbm, di_hbm, alpha_hbm, z_hbm, geom_hbm, sph_hbm,
          si_v, di_v, rs_v, rd_v, mb, pb0, pb1, pb2, pb3, alpha_v, acc,
          sem1, sem2):
        cid = lax.axis_index("c")
        sid = lax.axis_index("s")
        wid = cid * 16 + sid
        pltpu.sync_copy(alpha_hbm, alpha_v)
        pltpu.sync_copy(z_hbm, acc.at[pl.ds(sid * rows_per_sub, rows_per_sub)])
        na0 = alpha_v[pl.ds(0, 16)]
        na1 = alpha_v[pl.ds(16, 16)]
        iota = lax.iota(i32, 16)
        zeros16 = jnp.zeros((16,), f32)
        for cc in range(4, 8):
            ccv = jnp.full((16,), cc, i32)
            for g in range(8):
                plsc.store_scatter(mb, [g * 16 + iota, ccv], zeros16)
        plsc.subcore_barrier()
        base = wid * EPW
        pbs = (pb0, pb1, pb2, pb3)

        def group(g, poff):
            rowi = g * 16 + iota
            cid8 = [jnp.full((16,), cc, i32) for cc in range(7)]
            ps = [plsc.load_gather(rs_v, [rowi, cid8[cc]]) for cc in range(7)]
            pd = [plsc.load_gather(rd_v, [rowi, cid8[cc]]) for cc in range(3)]
            vx = pd[0] - ps[0]
            vy = pd[1] - ps[1]
            vz = pd[2] - ps[2]
            d2 = vx * vx + vy * vy + vz * vz + 1e-12
            ii = plsc.bitcast(d2, i32)
            y = plsc.bitcast(1597463007 - jnp.right_shift(ii, 1), f32)
            for _ in range(3):
                y = y * (1.5 - (0.5 * d2) * (y * y))
            dist = d2 * y
            ux = vx * y
            uy = vy * y
            uz = vz * y
            t = dist * (1.0 / CUTOFF)
            t2 = t * t
            t3 = t2 * t
            t6 = t3 * t3
            t7 = t6 * t
            t8 = t7 * t
            fc = 1.0 - 28.0 * t6 + 48.0 * t7 - 21.0 * t8
            fc = jnp.where(t < 1.0, fc, jnp.zeros((16,), f32))
            rad0 = jnp.exp(na0 * d2) * fc
            g1 = (SQRT3 * jnp.exp(na1 * d2)) * fc
            plsc.store_scatter(mb, [rowi, cid8[0]], ps[3] * rad0)
            plsc.store_scatter(mb, [rowi, cid8[1]], ps[4] * (g1 * ux))
            plsc.store_scatter(mb, [rowi, cid8[2]], ps[5] * (g1 * uy))
            plsc.store_scatter(mb, [rowi, cid8[3]], ps[6] * (g1 * uz))
            off = poff + g * 16
            pb0[pl.ds(off, 16)] = dist
            pb1[pl.ds(off, 16)] = ux
            pb2[pl.ds(off, 16)] = uy
            pb3[pl.ds(off, 16)] = uz

        @pl.loop(0, NSUP)
        def _(sj):
            pltpu.sync_copy(si_hbm.at[wid, pl.ds(sj * 8, 8)], si_v)
            pltpu.sync_copy(di_hbm.at[wid, pl.ds(sj * 8, 8)], di_v)

            @pl.loop(0, 8)
            def _(jj):
                a = pltpu.async_copy(T_hbm.at[si_v.at[jj]], rs_v, sem1)
                b = pltpu.async_copy(T_hbm.at[di_v.at[jj]], rd_v, sem2)
                a.wait()
                b.wait()
                poff = jj * 128
                for g in range(8):
                    group(g, poff)
                pltpu.sync_copy(mb, acc.at[di_v.at[jj]], add=True)

            sbase = base + sj * 1024
            for cp in range(4):
                pltpu.sync_copy(pbs[cp], geom_hbm.at[cp, pl.ds(sbase, 1024)])

        plsc.subcore_barrier()

        @pl.when(sid == 0)
        def _():
            pltpu.sync_copy(acc, sph_hbm.at[cid])

    return k(T, src_r, dst_r, gto_alpha, zblk)


def _erbf_body(geom_ref, erbf_ref):
    d = geom_ref[0]
    invd = 1.0 / d
    t = d * (1.0 / CUTOFF)
    t2 = t * t
    t3 = t2 * t
    t6 = t3 * t3
    t7 = t6 * t
    t8 = t7 * t
    fc = 1.0 - 28.0 * t6 + 48.0 * t7 - 21.0 * t8
    fc = jnp.where(t < 1.0, fc, 0.0)
    theta = d * (jnp.pi / CUTOFF)
    s1 = jnp.sin(theta)
    c2 = 2.0 * jnp.cos(theta)
    pf = jnp.sqrt(2.0 / CUTOFF) * fc * invd
    sm2 = jnp.zeros_like(s1)
    sm1 = s1
    erbf_ref[0] = sm1 * pf
    for n in range(1, NBASIS):
        sn = c2 * sm1 - sm2
        sm2 = sm1
        sm1 = sn
        erbf_ref[n] = sn * pf


def _tc_erbf(geom_r):
    BR = 64
    ner = NE_PAD // 128
    return pl.pallas_call(
        _erbf_body,
        grid=(ner // BR,),
        in_specs=[pl.BlockSpec((1, BR, 128), lambda i: (0, i, 0))],
        out_specs=pl.BlockSpec((NBASIS, BR, 128), lambda i: (0, i, 0)),
        out_shape=jax.ShapeDtypeStruct((NBASIS, ner, 128), jnp.float32),
    )(geom_r)


def _br(a):
    return a.astype(jnp.bfloat16).astype(jnp.float32)


def _node_math(sph, w_ref, W01_ref, W11_ref, W02_ref, W12_ref):
    s = sph[:, 0:1]
    vx = sph[:, 1:2]
    vy = sph[:, 2:3]
    vz = sph[:, 3:4]
    w0 = w_ref[0]
    w1 = w_ref[1]
    w2 = w_ref[2]
    w3 = w_ref[3]
    o0a = w0 * s * s
    o0b = (w1 / SQRT3) * (vx * vx + vy * vy + vz * vz)
    W01 = _br(W01_ref[...] * (1.0 / jnp.sqrt(2.0)))
    ns = _br(o0a) * W01[0:1, :] + _br(o0b) * W01[1:2, :]
    W11 = _br(W11_ref[...] * (1.0 / jnp.sqrt(2.0)))
    sv = s
    nvx = _br(w2 * sv * vx) * W11[0:1, :] + _br(w3 * sv * vx) * W11[1:2, :]
    nvy = _br(w2 * sv * vy) * W11[0:1, :] + _br(w3 * sv * vy) * W11[1:2, :]
    nvz = _br(w2 * sv * vz) * W11[0:1, :] + _br(w3 * sv * vz) * W11[1:2, :]
    ns = jax.nn.sigmoid(ns)
    vnorm = jnp.sqrt(nvx * nvx + nvy * nvy + nvz * nvz + 1e-12)
    gate = jax.nn.sigmoid(vnorm)
    nvx = nvx * gate
    nvy = nvy * gate
    nvz = nvz * gate
    bf = jnp.bfloat16
    f32 = jnp.float32
    W02 = (W02_ref[...] * (1.0 / jnp.sqrt(128.0))).astype(bf)
    ns2 = jnp.dot(ns.astype(bf), W02, preferred_element_type=f32)
    W12 = (W12_ref[...] * (1.0 / 8.0)).astype(bf)
    nvx2 = jnp.dot(nvx.astype(bf), W12, preferred_element_type=f32)
    nvy2 = jnp.dot(nvy.astype(bf), W12, preferred_element_type=f32)
    nvz2 = jnp.dot(nvz.astype(bf), W12, preferred_element_type=f32)
    return ns2, nvx2, nvy2, nvz2


BN = 2000


def _stats_body(sph_ref, w_ref, W01_ref, W11_ref, W02_ref, W12_ref, st_ref):
    sph = sph_ref[0] + sph_ref[1]
    ns2, nvx2, nvy2, nvz2 = _node_math(sph, w_ref, W01_ref, W11_ref,
                                       W02_ref, W12_ref)
    ssum = jnp.sum(ns2, axis=0).reshape(1, 128)
    ssq = jnp.sum(ns2 * ns2, axis=0).reshape(1, 128)
    vn2 = jnp.sum(nvx2 * nvx2 + nvy2 * nvy2 + nvz2 * nvz2, axis=0)
    vn2 = jnp.concatenate([vn2, jnp.zeros((64,), jnp.float32)]).reshape(1, 128)
    contrib = jnp.concatenate(
        [ssum, ssq, vn2, jnp.zeros((5, 128), jnp.float32)], axis=0)

    @pl.when(pl.program_id(0) == 0)
    def _():
        st_ref[...] = jnp.zeros_like(st_ref)

    st_ref[...] += contrib


def _norm_body(sph_ref, st_ref, w_ref, W01_ref, W11_ref, W02_ref, W12_ref,
               gs_ref, bs_ref, gv_ref, ns_ref, nvx_ref, nvy_ref, nvz_ref):
    sph = sph_ref[0] + sph_ref[1]
    ns2, nvx2, nvy2, nvz2 = _node_math(sph, w_ref, W01_ref, W11_ref,
                                       W02_ref, W12_ref)
    st = st_ref[...]
    inv_n = 1.0 / N_NODES
    mean = st[0:1, :] * inv_n
    var = st[1:2, :] * inv_n - mean * mean
    scale = gs_ref[...] / jnp.sqrt(var + 1e-5)
    ns_ref[...] = (ns2 - mean) * scale + bs_ref[...]
    vn2m = st[2:3, 0:64] * inv_n
    vfac = gv_ref[...] / jnp.sqrt(vn2m + 1e-5)
    nvx_ref[...] = nvx2 * vfac
    nvy_ref[...] = nvy2 * vfac
    nvz_ref[...] = nvz2 * vfac


def _tc_node(parts, w_self, W0_1, W1_1, W0_2, W1_2, gamma_s, beta_s, gamma_v):
    nb = N_NODES // BN
    wspec = [
        pl.BlockSpec(memory_space=pltpu.SMEM),
        pl.BlockSpec((2, 128), lambda i: (0, 0)),
        pl.BlockSpec((2, 64), lambda i: (0, 0)),
        pl.BlockSpec((128, 128), lambda i: (0, 0)),
        pl.BlockSpec((64, 64), lambda i: (0, 0)),
    ]
    sph_spec = pl.BlockSpec((2, BN, 8), lambda i: (0, i, 0))
    stats = pl.pallas_call(
        _stats_body,
        grid=(nb,),
        in_specs=[sph_spec] + wspec,
        out_specs=pl.BlockSpec((8, 128), lambda i: (0, 0)),
        out_shape=jax.ShapeDtypeStruct((8, 128), jnp.float32),
    )(parts, w_self, W0_1, W1_1, W0_2, W1_2)
    ns, nvx, nvy, nvz = pl.pallas_call(
        _norm_body,
        grid=(nb,),
        in_specs=[sph_spec, pl.BlockSpec((8, 128), lambda i: (0, 0))] + wspec
        + [pl.BlockSpec((1, 128), lambda i: (0, 0)),
           pl.BlockSpec((1, 128), lambda i: (0, 0)),
           pl.BlockSpec((1, 64), lambda i: (0, 0))],
        out_specs=[
            pl.BlockSpec((BN, 128), lambda i: (i, 0)),
            pl.BlockSpec((BN, 64), lambda i: (i, 0)),
            pl.BlockSpec((BN, 64), lambda i: (i, 0)),
            pl.BlockSpec((BN, 64), lambda i: (i, 0)),
        ],
        out_shape=[
            jax.ShapeDtypeStruct((N_NODES, 128), jnp.float32),
            jax.ShapeDtypeStruct((N_NODES, 64), jnp.float32),
            jax.ShapeDtypeStruct((N_NODES, 64), jnp.float32),
            jax.ShapeDtypeStruct((N_NODES, 64), jnp.float32),
        ],
    )(parts, stats, w_self, W0_1, W1_1, W0_2, W1_2,
      gamma_s.reshape(1, 128), beta_s.reshape(1, 128), gamma_v.reshape(1, 64))
    return ns, nvx, nvy, nvz


def kernel(x, pos, edge_index, w_self, W0_1, W1_1, W0_2, W1_2,
           gamma_s, beta_s, gamma_v, gto_alpha):
    f32 = jnp.float32
    pos_p = pos[:, jnp.array([1, 2, 0])]
    T = jnp.zeros((N_PAD, 8), f32)
    T = T.at[:N_NODES, 0:3].set(pos_p).at[:N_NODES, 3:7].set(x)
    npad = NE_PAD - N_EDGES
    src = jnp.concatenate([edge_index[0], jnp.zeros((npad,), jnp.int32)])
    dst = jnp.concatenate([edge_index[1],
                           jnp.full((npad,), DUMMY, jnp.int32)])
    src_r = src.reshape(NW, NCH, C)
    dst_r = dst.reshape(NW, NCH, C)
    zblk = jnp.zeros((N_PAD // 16, 8), f32)
    alpha32 = jnp.repeat(-gto_alpha, 16)

    geom, parts = _sc_edge(T, src_r, dst_r, alpha32, zblk)

    geom_r = geom.reshape(4, NE_PAD // 128, 128)
    erbf_p = _tc_erbf(geom_r)
    erbf = erbf_p.reshape(NBASIS, NE_PAD)[:, :N_EDGES].T
    u_t = geom[1:4, :N_EDGES].T
    ersh = jnp.concatenate(
        [jnp.ones((N_EDGES, 1), f32), -SQRT3 * u_t], axis=1)

    ns, nvx, nvy, nvz = _tc_node(parts, w_self, W0_1, W1_1, W0_2, W1_2,
                                 gamma_s, beta_s, gamma_v)
    nv = jnp.stack([nvx, nvy, nvz], axis=-1).reshape(N_NODES, 192)
    node = jnp.concatenate([ns, nv], axis=1)
    return node, erbf, ersh
